# baseline (speedup 1.0000x reference)
import functools

import jax
import jax.numpy as jnp
from jax import lax
from jax.experimental import pallas as pl
from jax.experimental.pallas import tpu as pltpu
from jax.experimental.pallas import tpu_sc as plsc

N = 10000
NP = 10240
D = 128
H = 64
E = 320000
NC = 2
NS = 16
NW = NC * NS
EW = E // NW
K = 125
NCH = EW // K
RPS = NP // NS

_mesh = plsc.VectorSubcoreMesh(
    core_axis_name="c", subcore_axis_name="s", num_cores=NC, num_subcores=NS
)
_sc_params = pltpu.CompilerParams(use_tc_tiling_on_sc=False)


@functools.partial(
    pl.kernel,
    out_type=jax.ShapeDtypeStruct((NC, NP), jnp.float32),
    mesh=_mesh,
    scratch_types=[
        pltpu.VMEM((NCH, K), jnp.int32),
        pltpu.VMEM((K,), jnp.float32),
        pltpu.VMEM_SHARED((NP,), jnp.float32),
        pltpu.SemaphoreType.DMA,
    ],
    compiler_params=_sc_params,
)
def _deg_kernel(dst3_hbm, zeros1_hbm, ones_hbm, degp_hbm, didx, ones_v, sdeg,
                sem):
    c = lax.axis_index("c")
    s = lax.axis_index("s")
    w = c * NS + s
    pltpu.sync_copy(dst3_hbm.at[w], didx)
    pltpu.sync_copy(zeros1_hbm, sdeg.at[pl.ds(s * RPS, RPS)])
    pltpu.sync_copy(ones_hbm, ones_v)
    plsc.subcore_barrier()

    def body(j, carry):
        descs = [
            pltpu.async_copy(ones_v, sdeg.at[didx.at[5 * j + i]], sem, add=True)
            for i in range(5)
        ]
        for d in descs:
            d.wait()
        return carry

    lax.fori_loop(0, NCH // 5, body, 0)
    plsc.subcore_barrier()
    pltpu.sync_copy(
        sdeg.at[pl.ds(s * RPS, RPS)], degp_hbm.at[c].at[pl.ds(s * RPS, RPS)]
    )


@functools.partial(
    pl.kernel,
    out_type=jax.ShapeDtypeStruct((NC, NP, H), jnp.float32),
    mesh=_mesh,
    scratch_types=[
        pltpu.VMEM((NCH, K), jnp.int32),
        pltpu.VMEM((NCH, K), jnp.int32),
        pltpu.VMEM((8, K, H), jnp.float32),
        pltpu.VMEM_SHARED((NP, H), jnp.float32),
        pltpu.SemaphoreType.DMA((8,)),
        pltpu.SemaphoreType.DMA((8,)),
    ],
    compiler_params=_sc_params,
)
def _scatter_kernel(hws_hbm, src3_hbm, dst3_hbm, zeros2_hbm, part_hbm,
                    sidx, didx, rows, acc, gsem, ssem):
    c = lax.axis_index("c")
    s = lax.axis_index("s")
    w = c * NS + s
    pltpu.sync_copy(src3_hbm.at[w], sidx)
    pltpu.sync_copy(dst3_hbm.at[w], didx)
    pltpu.sync_copy(zeros2_hbm, acc.at[pl.ds(s * RPS, RPS)])
    plsc.subcore_barrier()

    UNR = 4
    NG = NCH // UNR

    def _fire_gather(g, slot):
        for i in range(UNR):
            pltpu.async_copy(
                hws_hbm.at[sidx.at[g * UNR + i]], rows.at[slot + i],
                gsem.at[slot + i],
            )

    def _wait_gather_fire_scatter(g, slot):
        for i in range(UNR):
            pltpu.make_async_copy(
                hws_hbm.at[sidx.at[g * UNR + i]], rows.at[slot + i],
                gsem.at[slot + i],
            ).wait()
            pltpu.async_copy(
                rows.at[slot + i], acc.at[didx.at[g * UNR + i]],
                ssem.at[slot + i], add=True,
            )

    def _wait_scatter(g, slot):
        for i in range(UNR):
            pltpu.make_async_copy(
                rows.at[slot + i], acc.at[didx.at[g * UNR + i]],
                ssem.at[slot + i],
            ).wait()

    _fire_gather(0, 0)
    _fire_gather(1, UNR)

    def body(m, carry):
        ga = 2 * m
        _wait_gather_fire_scatter(ga, 0)
        _wait_scatter(ga, 0)
        _fire_gather(ga + 2, 0)
        _wait_gather_fire_scatter(ga + 1, UNR)
        _wait_scatter(ga + 1, UNR)
        _fire_gather(ga + 3, UNR)
        return carry

    lax.fori_loop(0, NG // 2 - 1, body, 0)
    _wait_gather_fire_scatter(NG - 2, 0)
    _wait_gather_fire_scatter(NG - 1, UNR)
    _wait_scatter(NG - 2, 0)
    _wait_scatter(NG - 1, UNR)
    plsc.subcore_barrier()
    pltpu.sync_copy(
        acc.at[pl.ds(s * RPS, RPS)], part_hbm.at[c].at[pl.ds(s * RPS, RPS)]
    )


def _dinv_from(degp):
    deg = degp[0, :N] + degp[1, :N] + 1.0
    return lax.rsqrt(jnp.clip(deg, 1.0))


NB = 10
BR = N // NB
_HI = lax.Precision.HIGHEST


def _dinv_blk(degp):
    deg = degp[0, :, 0] + degp[1, :, 0] + 1.0
    return lax.rsqrt(jnp.clip(deg, 1.0))[:, None]


def _tc0_body(degp_ref, x_ref, w0_ref, hws_ref):
    dinv = _dinv_blk(degp_ref[...])
    hw = jnp.dot(x_ref[...], w0_ref[...], preferred_element_type=jnp.float32,
                 precision=_HI)
    hws_ref[...] = dinv * hw


_tc0 = pl.pallas_call(
    _tc0_body,
    grid=(NB,),
    in_specs=[
        pl.BlockSpec((NC, BR, 1), lambda b: (0, b, 0)),
        pl.BlockSpec((BR, D), lambda b: (b, 0)),
        pl.BlockSpec((D, H), lambda b: (0, 0)),
    ],
    out_specs=pl.BlockSpec((BR, H), lambda b: (b, 0)),
    out_shape=jax.ShapeDtypeStruct((N, H), jnp.float32),
)


def _bn_pass0(part_ref, hws_ref, degp_ref, b_ref, pre_sc, stat_sc, blk):
    dinv = _dinv_blk(degp_ref[...])
    S = part_ref[0] + part_ref[1]
    pre = dinv * (S + hws_ref[...]) + b_ref[...][None, :]
    pre_sc[pl.ds(blk * BR, BR), :] = pre

    @pl.when(blk == 0)
    def _():
        stat_sc[...] = jnp.zeros_like(stat_sc)

    stat_sc[0, :] += jnp.sum(pre, axis=0)
    stat_sc[1, :] += jnp.sum(pre * pre, axis=0)


def _bn_norm(pre_sc, stat_sc, g_ref, be_ref, blk):
    mu = stat_sc[0, :] * (1.0 / N)
    var = stat_sc[1, :] * (1.0 / N) - mu * mu
    pre = pre_sc[pl.ds(blk * BR, BR), :]
    hb = (pre - mu[None, :]) / jnp.sqrt(var + 1e-5) * g_ref[...][None, :] \
        + be_ref[...][None, :]
    return jnp.maximum(hb, 0.0)


def _mid_body(part_ref, hws_ref, degp_ref, b_ref, g_ref, be_ref, wn_ref,
              out_ref, pre_sc, stat_sc):
    p = pl.program_id(0)
    blk = pl.program_id(1)

    @pl.when(p == 0)
    def _():
        _bn_pass0(part_ref, hws_ref, degp_ref, b_ref, pre_sc, stat_sc, blk)

    @pl.when(p == 1)
    def _():
        h = _bn_norm(pre_sc, stat_sc, g_ref, be_ref, blk)
        dinv = _dinv_blk(degp_ref[...])
        out_ref[...] = dinv * jnp.dot(
            h, wn_ref[...], preferred_element_type=jnp.float32, precision=_HI
        )


_mid = pl.pallas_call(
    _mid_body,
    grid=(2, NB),
    in_specs=[
        pl.BlockSpec((NC, BR, H), lambda p, b: (0, (1 - p) * b, 0)),
        pl.BlockSpec((BR, H), lambda p, b: ((1 - p) * b, 0)),
        pl.BlockSpec((NC, BR, 1), lambda p, b: (0, b, 0)),
        pl.BlockSpec((H,), lambda p, b: (0,)),
        pl.BlockSpec((H,), lambda p, b: (0,)),
        pl.BlockSpec((H,), lambda p, b: (0,)),
        pl.BlockSpec((H, H), lambda p, b: (0, 0)),
    ],
    out_specs=pl.BlockSpec((BR, H), lambda p, b: (b, 0)),
    out_shape=jax.ShapeDtypeStruct((N, H), jnp.float32),
    scratch_shapes=[
        pltpu.VMEM((N, H), jnp.float32),
        pltpu.VMEM((2, H), jnp.float32),
    ],
)


def _fin_body(part_ref, hws_ref, degp_ref, b_ref, g_ref, be_ref,
              fc1w_ref, fc1b_ref, fc2w_ref, fc2b_ref, out_ref,
              pre_sc, stat_sc):
    p = pl.program_id(0)
    blk = pl.program_id(1)

    @pl.when(p == 0)
    def _():
        _bn_pass0(part_ref, hws_ref, degp_ref, b_ref, pre_sc, stat_sc, blk)

    @pl.when(p == 1)
    def _():
        h = _bn_norm(pre_sc, stat_sc, g_ref, be_ref, blk)
        z = jnp.maximum(
            jnp.dot(h, fc1w_ref[...], preferred_element_type=jnp.float32,
                    precision=_HI) + fc1b_ref[...][None, :],
            0.0,
        )
        o = (
            jnp.dot(z, fc2w_ref[...], preferred_element_type=jnp.float32,
                    precision=_HI) + fc2b_ref[...][None, :]
        )
        m = jnp.max(o, axis=1, keepdims=True)
        lse = jnp.log(jnp.sum(jnp.exp(o - m), axis=1, keepdims=True)) + m
        out_ref[...] = o - lse


_fin = pl.pallas_call(
    _fin_body,
    grid=(2, NB),
    in_specs=[
        pl.BlockSpec((NC, BR, H), lambda p, b: (0, (1 - p) * b, 0)),
        pl.BlockSpec((BR, H), lambda p, b: ((1 - p) * b, 0)),
        pl.BlockSpec((NC, BR, 1), lambda p, b: (0, b, 0)),
        pl.BlockSpec((H,), lambda p, b: (0,)),
        pl.BlockSpec((H,), lambda p, b: (0,)),
        pl.BlockSpec((H,), lambda p, b: (0,)),
        pl.BlockSpec((H, 32), lambda p, b: (0, 0)),
        pl.BlockSpec((32,), lambda p, b: (0,)),
        pl.BlockSpec((32, 2), lambda p, b: (0, 0)),
        pl.BlockSpec((2,), lambda p, b: (0,)),
    ],
    out_specs=pl.BlockSpec((BR, 2), lambda p, b: (b, 0)),
    out_shape=jax.ShapeDtypeStruct((N, 2), jnp.float32),
    scratch_shapes=[
        pltpu.VMEM((N, H), jnp.float32),
        pltpu.VMEM((2, H), jnp.float32),
    ],
)


def kernel(x, edge_index, W0, b0, W1, b1, W2, b2, g0, be0, g1, be1, g2, be2,
           fc1_w, fc1_b, fc2_w, fc2_b):
    src = edge_index[0]
    dst = edge_index[1]
    src3 = src.reshape(NW, NCH, K)
    dst3 = dst.reshape(NW, NCH, K)
    zeros1 = jnp.zeros((RPS,), jnp.float32)
    zeros2 = jnp.zeros((RPS, H), jnp.float32)
    ones = jnp.ones((K,), jnp.float32)

    degp = _deg_kernel(dst3, zeros1, ones)[..., None]
    hws = _tc0(degp, x, W0)
    for b, g, be, Wn in ((b0, g0, be0, W1), (b1, g1, be1, W2)):
        part = _scatter_kernel(hws, src3, dst3, zeros2)
        hws = _mid(part, hws, degp, b, g, be, Wn)
    part = _scatter_kernel(hws, src3, dst3, zeros2)
    return _fin(part, hws, degp, b2, g2, be2, fc1_w, fc1_b, fc2_w, fc2_b)


# speedup vs baseline: 1.1534x; 1.1534x over previous
import functools

import jax
import jax.numpy as jnp
from jax import lax
from jax.experimental import pallas as pl
from jax.experimental.pallas import tpu as pltpu
from jax.experimental.pallas import tpu_sc as plsc

N = 10000
NP = 10240
D = 128
H = 64
E = 320000
NC = 2
NS = 16
NW = NC * NS
EW = E // NW
K = 125
NCH = EW // K
RPS = NP // NS

_mesh = plsc.VectorSubcoreMesh(
    core_axis_name="c", subcore_axis_name="s", num_cores=NC, num_subcores=NS
)
_sc_params = pltpu.CompilerParams(use_tc_tiling_on_sc=False)


@functools.partial(
    pl.kernel,
    out_type=jax.ShapeDtypeStruct((NC, NP), jnp.float32),
    mesh=_mesh,
    scratch_types=[
        pltpu.VMEM((NCH, K), jnp.int32),
        pltpu.VMEM((K,), jnp.float32),
        pltpu.VMEM_SHARED((NP,), jnp.float32),
        pltpu.SemaphoreType.DMA,
    ],
    compiler_params=_sc_params,
)
def _deg_kernel(dst3_hbm, zeros1_hbm, ones_hbm, degp_hbm, didx, ones_v, sdeg,
                sem):
    c = lax.axis_index("c")
    s = lax.axis_index("s")
    w = c * NS + s
    pltpu.sync_copy(dst3_hbm.at[w], didx)
    pltpu.sync_copy(zeros1_hbm, sdeg.at[pl.ds(s * RPS, RPS)])
    pltpu.sync_copy(ones_hbm, ones_v)
    plsc.subcore_barrier()

    def body(j, carry):
        descs = [
            pltpu.async_copy(ones_v, sdeg.at[didx.at[5 * j + i]], sem, add=True)
            for i in range(5)
        ]
        for d in descs:
            d.wait()
        return carry

    lax.fori_loop(0, NCH // 5, body, 0)
    plsc.subcore_barrier()
    pltpu.sync_copy(
        sdeg.at[pl.ds(s * RPS, RPS)], degp_hbm.at[c].at[pl.ds(s * RPS, RPS)]
    )


@functools.partial(
    pl.kernel,
    out_type=jax.ShapeDtypeStruct((NC, NP, H), jnp.float32),
    mesh=_mesh,
    scratch_types=[
        pltpu.VMEM((NCH, K), jnp.int32),
        pltpu.VMEM((NCH, K), jnp.int32),
        pltpu.VMEM((8, K, H), jnp.float32),
        pltpu.VMEM_SHARED((NP, H), jnp.float32),
        pltpu.SemaphoreType.DMA((8,)),
        pltpu.SemaphoreType.DMA((8,)),
    ],
    compiler_params=_sc_params,
)
def _scatter_kernel(hws_hbm, src3_hbm, dst3_hbm, zeros2_hbm, part_hbm,
                    sidx, didx, rows, acc, gsem, ssem):
    c = lax.axis_index("c")
    s = lax.axis_index("s")
    w = c * NS + s
    pltpu.sync_copy(src3_hbm.at[w], sidx)
    pltpu.sync_copy(dst3_hbm.at[w], didx)
    pltpu.sync_copy(zeros2_hbm, acc.at[pl.ds(s * RPS, RPS)])
    plsc.subcore_barrier()

    UNR = 4
    NG = NCH // UNR

    def _fire_gather(g, slot):
        for i in range(UNR):
            pltpu.async_copy(
                hws_hbm.at[sidx.at[g * UNR + i]], rows.at[slot + i],
                gsem.at[slot + i],
            )

    def _wait_gather_fire_scatter(g, slot):
        for i in range(UNR):
            pltpu.make_async_copy(
                hws_hbm.at[sidx.at[g * UNR + i]], rows.at[slot + i],
                gsem.at[slot + i],
            ).wait()
            pltpu.async_copy(
                rows.at[slot + i], acc.at[didx.at[g * UNR + i]],
                ssem.at[slot + i], add=True,
            )

    def _wait_scatter(g, slot):
        for i in range(UNR):
            pltpu.make_async_copy(
                rows.at[slot + i], acc.at[didx.at[g * UNR + i]],
                ssem.at[slot + i],
            ).wait()

    _fire_gather(0, 0)
    _fire_gather(1, UNR)

    def body(m, carry):
        ga = 2 * m
        _wait_gather_fire_scatter(ga, 0)
        _wait_scatter(ga, 0)
        _fire_gather(ga + 2, 0)
        _wait_gather_fire_scatter(ga + 1, UNR)
        _wait_scatter(ga + 1, UNR)
        _fire_gather(ga + 3, UNR)
        return carry

    lax.fori_loop(0, NG // 2 - 1, body, 0)
    _wait_gather_fire_scatter(NG - 2, 0)
    _wait_gather_fire_scatter(NG - 1, UNR)
    _wait_scatter(NG - 2, 0)
    _wait_scatter(NG - 1, UNR)
    plsc.subcore_barrier()
    pltpu.sync_copy(
        acc.at[pl.ds(s * RPS, RPS)], part_hbm.at[c].at[pl.ds(s * RPS, RPS)]
    )


_HI = lax.Precision.HIGHEST


def _dinv_from(degp):
    deg = degp[0, :N] + degp[1, :N] + 1.0
    return lax.rsqrt(jnp.clip(deg, 1.0))


def _tc0_body(degp_ref, x_ref, w0_ref, hws_ref):
    dinv = _dinv_from(degp_ref[...])
    hw = jnp.dot(x_ref[...], w0_ref[...], preferred_element_type=jnp.float32,
                 precision=_HI)
    hws_ref[...] = dinv[:, None] * hw


_tc0 = pl.pallas_call(
    _tc0_body, out_shape=jax.ShapeDtypeStruct((N, H), jnp.float32)
)


def _bn_relu(part, hws, dinv, b, g, be):
    S = part[0, :N, :] + part[1, :N, :]
    pre = dinv * (S + hws) + b[None, :]
    mu = jnp.mean(pre, axis=0)
    var = jnp.mean((pre - mu[None, :]) ** 2, axis=0)
    hb = (pre - mu[None, :]) / jnp.sqrt(var + 1e-5) * g[None, :] + be[None, :]
    return jnp.maximum(hb, 0.0)


def _mid_body(part_ref, hws_ref, degp_ref, b_ref, g_ref, be_ref, wn_ref,
              out_ref):
    dinv = _dinv_from(degp_ref[...])[:, None]
    h = _bn_relu(part_ref[...], hws_ref[...], dinv, b_ref[...], g_ref[...],
                 be_ref[...])
    out_ref[...] = dinv * jnp.dot(
        h, wn_ref[...], preferred_element_type=jnp.float32, precision=_HI
    )


_mid = pl.pallas_call(
    _mid_body, out_shape=jax.ShapeDtypeStruct((N, H), jnp.float32)
)


def _fin_body(part_ref, hws_ref, degp_ref, b_ref, g_ref, be_ref,
              fc1w_ref, fc1b_ref, fc2w_ref, fc2b_ref, out_ref):
    dinv = _dinv_from(degp_ref[...])[:, None]
    h = _bn_relu(part_ref[...], hws_ref[...], dinv, b_ref[...], g_ref[...],
                 be_ref[...])
    z = jnp.maximum(
        jnp.dot(h, fc1w_ref[...], preferred_element_type=jnp.float32,
                precision=_HI) + fc1b_ref[...][None, :],
        0.0,
    )
    o = (
        jnp.dot(z, fc2w_ref[...], preferred_element_type=jnp.float32,
                precision=_HI) + fc2b_ref[...][None, :]
    )
    m = jnp.max(o, axis=1, keepdims=True)
    lse = jnp.log(jnp.sum(jnp.exp(o - m), axis=1, keepdims=True)) + m
    out_ref[...] = o - lse


_fin = pl.pallas_call(
    _fin_body, out_shape=jax.ShapeDtypeStruct((N, 2), jnp.float32)
)


def kernel(x, edge_index, W0, b0, W1, b1, W2, b2, g0, be0, g1, be1, g2, be2,
           fc1_w, fc1_b, fc2_w, fc2_b):
    src = edge_index[0]
    dst = edge_index[1]
    src3 = src.reshape(NW, NCH, K)
    dst3 = dst.reshape(NW, NCH, K)
    zeros1 = jnp.zeros((RPS,), jnp.float32)
    zeros2 = jnp.zeros((RPS, H), jnp.float32)
    ones = jnp.ones((K,), jnp.float32)

    degp = _deg_kernel(dst3, zeros1, ones)
    hws = _tc0(degp, x, W0)
    for b, g, be, Wn in ((b0, g0, be0, W1), (b1, g1, be1, W2)):
        part = _scatter_kernel(hws, src3, dst3, zeros2)
        hws = _mid(part, hws, degp, b, g, be, Wn)
    part = _scatter_kernel(hws, src3, dst3, zeros2)
    return _fin(part, hws, degp, b2, g2, be2, fc1_w, fc1_b, fc2_w, fc2_b)


# speedup vs baseline: 1.1877x; 1.0297x over previous
import functools

import jax
import jax.numpy as jnp
from jax import lax
from jax.experimental import pallas as pl
from jax.experimental.pallas import tpu as pltpu
from jax.experimental.pallas import tpu_sc as plsc

N = 10000
NP = 10240
D = 128
H = 64
E = 320000
NC = 2
NS = 16
NW = NC * NS
EW = E // NW
K = 125
NCH = EW // K
RPS = NP // NS

_mesh = plsc.VectorSubcoreMesh(
    core_axis_name="c", subcore_axis_name="s", num_cores=NC, num_subcores=NS
)
_sc_params = pltpu.CompilerParams(use_tc_tiling_on_sc=False)


@functools.partial(
    pl.kernel,
    out_type=jax.ShapeDtypeStruct((NC, NP), jnp.float32),
    mesh=_mesh,
    scratch_types=[
        pltpu.VMEM((NCH, K), jnp.int32),
        pltpu.VMEM((K,), jnp.float32),
        pltpu.VMEM_SHARED((NP,), jnp.float32),
        pltpu.SemaphoreType.DMA,
    ],
    compiler_params=_sc_params,
)
def _deg_kernel(dst3_hbm, zeros1_hbm, ones_hbm, degp_hbm, didx, ones_v, sdeg,
                sem):
    c = lax.axis_index("c")
    s = lax.axis_index("s")
    w = c * NS + s
    pltpu.sync_copy(dst3_hbm.at[w], didx)
    pltpu.sync_copy(zeros1_hbm, sdeg.at[pl.ds(s * RPS, RPS)])
    pltpu.sync_copy(ones_hbm, ones_v)
    plsc.subcore_barrier()

    def body(j, carry):
        descs = [
            pltpu.async_copy(ones_v, sdeg.at[didx.at[5 * j + i]], sem, add=True)
            for i in range(5)
        ]
        for d in descs:
            d.wait()
        return carry

    lax.fori_loop(0, NCH // 5, body, 0)
    plsc.subcore_barrier()
    pltpu.sync_copy(
        sdeg.at[pl.ds(s * RPS, RPS)], degp_hbm.at[c].at[pl.ds(s * RPS, RPS)]
    )


@functools.partial(
    pl.kernel,
    out_type=jax.ShapeDtypeStruct((NC, NP, H), jnp.float32),
    mesh=_mesh,
    scratch_types=[
        pltpu.VMEM((NCH, K), jnp.int32),
        pltpu.VMEM((NCH, K), jnp.int32),
        pltpu.VMEM((8, K, H), jnp.float32),
        pltpu.VMEM_SHARED((NP, H), jnp.float32),
        pltpu.SemaphoreType.DMA((8,)),
        pltpu.SemaphoreType.DMA((8,)),
    ],
    compiler_params=_sc_params,
)
def _scatter_kernel(hws_hbm, src3_hbm, dst3_hbm, zeros2_hbm, part_hbm,
                    sidx, didx, rows, acc, gsem, ssem):
    c = lax.axis_index("c")
    s = lax.axis_index("s")
    w = c * NS + s
    pltpu.sync_copy(src3_hbm.at[w], sidx)
    pltpu.sync_copy(dst3_hbm.at[w], didx)

    @pl.when(c == 0)
    def _():
        pltpu.sync_copy(
            hws_hbm.at[pl.ds(s * RPS, RPS)], acc.at[pl.ds(s * RPS, RPS)]
        )

    @pl.when(c == 1)
    def _():
        pltpu.sync_copy(zeros2_hbm, acc.at[pl.ds(s * RPS, RPS)])

    plsc.subcore_barrier()

    UNR = 4
    NG = NCH // UNR

    def _fire_gather(g, slot):
        for i in range(UNR):
            pltpu.async_copy(
                hws_hbm.at[sidx.at[g * UNR + i]], rows.at[slot + i],
                gsem.at[slot + i],
            )

    def _wait_gather_fire_scatter(g, slot):
        for i in range(UNR):
            pltpu.make_async_copy(
                hws_hbm.at[sidx.at[g * UNR + i]], rows.at[slot + i],
                gsem.at[slot + i],
            ).wait()
            pltpu.async_copy(
                rows.at[slot + i], acc.at[didx.at[g * UNR + i]],
                ssem.at[slot + i], add=True,
            )

    def _wait_scatter(g, slot):
        for i in range(UNR):
            pltpu.make_async_copy(
                rows.at[slot + i], acc.at[didx.at[g * UNR + i]],
                ssem.at[slot + i],
            ).wait()

    _fire_gather(0, 0)
    _fire_gather(1, UNR)

    def body(m, carry):
        ga = 2 * m
        _wait_gather_fire_scatter(ga, 0)
        _wait_scatter(ga, 0)
        _fire_gather(ga + 2, 0)
        _wait_gather_fire_scatter(ga + 1, UNR)
        _wait_scatter(ga + 1, UNR)
        _fire_gather(ga + 3, UNR)
        return carry

    lax.fori_loop(0, NG // 2 - 1, body, 0)
    _wait_gather_fire_scatter(NG - 2, 0)
    _wait_gather_fire_scatter(NG - 1, UNR)
    _wait_scatter(NG - 2, 0)
    _wait_scatter(NG - 1, UNR)
    plsc.subcore_barrier()
    pltpu.sync_copy(
        acc.at[pl.ds(s * RPS, RPS)], part_hbm.at[c].at[pl.ds(s * RPS, RPS)]
    )


_HI = lax.Precision.HIGHEST


def _dinv_from(degp):
    deg = degp[0, :N] + degp[1, :N] + 1.0
    return lax.rsqrt(jnp.clip(deg, 1.0))


def _tc0_body(degp_ref, x_ref, w0_ref, hws_ref):
    dinv = _dinv_from(degp_ref[...])
    hw = jnp.dot(x_ref[...], w0_ref[...], preferred_element_type=jnp.float32,
                 precision=_HI)
    hws_ref[pl.ds(0, N), :] = dinv[:, None] * hw
    hws_ref[pl.ds(N, NP - N), :] = jnp.zeros((NP - N, H), jnp.float32)


_tc0 = pl.pallas_call(
    _tc0_body, out_shape=jax.ShapeDtypeStruct((NP, H), jnp.float32)
)


def _bn_relu(part, dinv, b, g, be):
    S = part[0, :N, :] + part[1, :N, :]
    pre = dinv * S + b[None, :]
    mu = jnp.mean(pre, axis=0)
    var = jnp.mean((pre - mu[None, :]) ** 2, axis=0)
    hb = (pre - mu[None, :]) / jnp.sqrt(var + 1e-5) * g[None, :] + be[None, :]
    return jnp.maximum(hb, 0.0)


def _mid_body(part_ref, degp_ref, b_ref, g_ref, be_ref, wn_ref, out_ref):
    dinv = _dinv_from(degp_ref[...])[:, None]
    h = _bn_relu(part_ref[...], dinv, b_ref[...], g_ref[...], be_ref[...])
    out_ref[pl.ds(0, N), :] = dinv * jnp.dot(
        h, wn_ref[...], preferred_element_type=jnp.float32, precision=_HI
    )
    out_ref[pl.ds(N, NP - N), :] = jnp.zeros((NP - N, H), jnp.float32)


_mid = pl.pallas_call(
    _mid_body, out_shape=jax.ShapeDtypeStruct((NP, H), jnp.float32)
)


def _fin_body(part_ref, degp_ref, b_ref, g_ref, be_ref,
              fc1w_ref, fc1b_ref, fc2w_ref, fc2b_ref, out_ref):
    dinv = _dinv_from(degp_ref[...])[:, None]
    h = _bn_relu(part_ref[...], dinv, b_ref[...], g_ref[...], be_ref[...])
    z = jnp.maximum(
        jnp.dot(h, fc1w_ref[...], preferred_element_type=jnp.float32,
                precision=_HI) + fc1b_ref[...][None, :],
        0.0,
    )
    o = (
        jnp.dot(z, fc2w_ref[...], preferred_element_type=jnp.float32,
                precision=_HI) + fc2b_ref[...][None, :]
    )
    m = jnp.max(o, axis=1, keepdims=True)
    lse = jnp.log(jnp.sum(jnp.exp(o - m), axis=1, keepdims=True)) + m
    out_ref[...] = o - lse


_fin = pl.pallas_call(
    _fin_body, out_shape=jax.ShapeDtypeStruct((N, 2), jnp.float32)
)


def kernel(x, edge_index, W0, b0, W1, b1, W2, b2, g0, be0, g1, be1, g2, be2,
           fc1_w, fc1_b, fc2_w, fc2_b):
    src = edge_index[0]
    dst = edge_index[1]
    src3 = src.reshape(NW, NCH, K)
    dst3 = dst.reshape(NW, NCH, K)
    zeros1 = jnp.zeros((RPS,), jnp.float32)
    zeros2 = jnp.zeros((RPS, H), jnp.float32)
    ones = jnp.ones((K,), jnp.float32)

    degp = _deg_kernel(dst3, zeros1, ones)
    hws = _tc0(degp, x, W0)
    for b, g, be, Wn in ((b0, g0, be0, W1), (b1, g1, be1, W2)):
        part = _scatter_kernel(hws, src3, dst3, zeros2)
        hws = _mid(part, degp, b, g, be, Wn)
    part = _scatter_kernel(hws, src3, dst3, zeros2)
    return _fin(part, degp, b2, g2, be2, fc1_w, fc1_b, fc2_w, fc2_b)


# speedup vs baseline: 1.2202x; 1.0273x over previous
import functools

import jax
import jax.numpy as jnp
from jax import lax
from jax.experimental import pallas as pl
from jax.experimental.pallas import tpu as pltpu
from jax.experimental.pallas import tpu_sc as plsc

N = 10000
NP = 10240
D = 128
H = 64
E = 320000
NC = 2
NS = 16
NW = NC * NS
EW = E // NW
K = 100
NCH = EW // K
RPS = NP // NS

_mesh = plsc.VectorSubcoreMesh(
    core_axis_name="c", subcore_axis_name="s", num_cores=NC, num_subcores=NS
)
_sc_params = pltpu.CompilerParams(use_tc_tiling_on_sc=False)


@functools.partial(
    pl.kernel,
    out_type=jax.ShapeDtypeStruct((NC, NP), jnp.float32),
    mesh=_mesh,
    scratch_types=[
        pltpu.VMEM((NCH, K), jnp.int32),
        pltpu.VMEM((K,), jnp.float32),
        pltpu.VMEM_SHARED((NP,), jnp.float32),
        pltpu.SemaphoreType.DMA,
    ],
    compiler_params=_sc_params,
)
def _deg_kernel(dst3_hbm, zeros1_hbm, ones_hbm, degp_hbm, didx, ones_v, sdeg,
                sem):
    c = lax.axis_index("c")
    s = lax.axis_index("s")
    w = c * NS + s
    pltpu.sync_copy(dst3_hbm.at[w], didx)
    pltpu.sync_copy(zeros1_hbm, sdeg.at[pl.ds(s * RPS, RPS)])
    pltpu.sync_copy(ones_hbm, ones_v)
    plsc.subcore_barrier()

    def body(j, carry):
        descs = [
            pltpu.async_copy(ones_v, sdeg.at[didx.at[5 * j + i]], sem, add=True)
            for i in range(5)
        ]
        for d in descs:
            d.wait()
        return carry

    lax.fori_loop(0, NCH // 5, body, 0)
    plsc.subcore_barrier()
    pltpu.sync_copy(
        sdeg.at[pl.ds(s * RPS, RPS)], degp_hbm.at[c].at[pl.ds(s * RPS, RPS)]
    )


@functools.partial(
    pl.kernel,
    out_type=jax.ShapeDtypeStruct((NC, NP, H), jnp.float32),
    mesh=_mesh,
    scratch_types=[
        pltpu.VMEM((NCH, K), jnp.int32),
        pltpu.VMEM((NCH, K), jnp.int32),
        pltpu.VMEM((10, K, H), jnp.float32),
        pltpu.VMEM_SHARED((NP, H), jnp.float32),
        pltpu.SemaphoreType.DMA((10,)),
        pltpu.SemaphoreType.DMA((10,)),
    ],
    compiler_params=_sc_params,
)
def _scatter_kernel(hws_hbm, src3_hbm, dst3_hbm, zeros2_hbm, part_hbm,
                    sidx, didx, rows, acc, gsem, ssem):
    c = lax.axis_index("c")
    s = lax.axis_index("s")
    w = c * NS + s
    pltpu.sync_copy(src3_hbm.at[w], sidx)
    pltpu.sync_copy(dst3_hbm.at[w], didx)

    @pl.when(c == 0)
    def _():
        pltpu.sync_copy(
            hws_hbm.at[pl.ds(s * RPS, RPS)], acc.at[pl.ds(s * RPS, RPS)]
        )

    @pl.when(c == 1)
    def _():
        pltpu.sync_copy(zeros2_hbm, acc.at[pl.ds(s * RPS, RPS)])

    plsc.subcore_barrier()

    UNR = 5
    NG = NCH // UNR

    def _fire_gather(g, slot):
        for i in range(UNR):
            pltpu.async_copy(
                hws_hbm.at[sidx.at[g * UNR + i]], rows.at[slot + i],
                gsem.at[slot + i],
            )

    def _wait_gather_fire_scatter(g, slot):
        for i in range(UNR):
            pltpu.make_async_copy(
                hws_hbm.at[sidx.at[g * UNR + i]], rows.at[slot + i],
                gsem.at[slot + i],
            ).wait()
            pltpu.async_copy(
                rows.at[slot + i], acc.at[didx.at[g * UNR + i]],
                ssem.at[slot + i], add=True,
            )

    def _wait_scatter(g, slot):
        for i in range(UNR):
            pltpu.make_async_copy(
                rows.at[slot + i], acc.at[didx.at[g * UNR + i]],
                ssem.at[slot + i],
            ).wait()

    _fire_gather(0, 0)
    _fire_gather(1, UNR)

    def body(m, carry):
        ga = 2 * m
        _wait_gather_fire_scatter(ga, 0)
        _wait_scatter(ga, 0)
        _fire_gather(ga + 2, 0)
        _wait_gather_fire_scatter(ga + 1, UNR)
        _wait_scatter(ga + 1, UNR)
        _fire_gather(ga + 3, UNR)
        return carry

    lax.fori_loop(0, NG // 2 - 1, body, 0)
    _wait_gather_fire_scatter(NG - 2, 0)
    _wait_gather_fire_scatter(NG - 1, UNR)
    _wait_scatter(NG - 2, 0)
    _wait_scatter(NG - 1, UNR)
    plsc.subcore_barrier()
    pltpu.sync_copy(
        acc.at[pl.ds(s * RPS, RPS)], part_hbm.at[c].at[pl.ds(s * RPS, RPS)]
    )


_HI = lax.Precision.HIGHEST


def _dinv_from(degp):
    deg = degp[0, :N] + degp[1, :N] + 1.0
    return lax.rsqrt(jnp.clip(deg, 1.0))


def _tc0_body(degp_ref, x_ref, w0_ref, hws_ref):
    dinv = _dinv_from(degp_ref[...])
    hw = jnp.dot(x_ref[...], w0_ref[...], preferred_element_type=jnp.float32,
                 precision=_HI)
    hws_ref[pl.ds(0, N), :] = dinv[:, None] * hw
    hws_ref[pl.ds(N, NP - N), :] = jnp.zeros((NP - N, H), jnp.float32)


_tc0 = pl.pallas_call(
    _tc0_body, out_shape=jax.ShapeDtypeStruct((NP, H), jnp.float32)
)


def _bn_relu(part, dinv, b, g, be):
    S = part[0, :N, :] + part[1, :N, :]
    pre = dinv * S + b[None, :]
    mu = jnp.mean(pre, axis=0)
    var = jnp.mean((pre - mu[None, :]) ** 2, axis=0)
    hb = (pre - mu[None, :]) / jnp.sqrt(var + 1e-5) * g[None, :] + be[None, :]
    return jnp.maximum(hb, 0.0)


def _mid_body(part_ref, degp_ref, b_ref, g_ref, be_ref, wn_ref, out_ref):
    dinv = _dinv_from(degp_ref[...])[:, None]
    h = _bn_relu(part_ref[...], dinv, b_ref[...], g_ref[...], be_ref[...])
    out_ref[pl.ds(0, N), :] = dinv * jnp.dot(
        h, wn_ref[...], preferred_element_type=jnp.float32, precision=_HI
    )
    out_ref[pl.ds(N, NP - N), :] = jnp.zeros((NP - N, H), jnp.float32)


_mid = pl.pallas_call(
    _mid_body, out_shape=jax.ShapeDtypeStruct((NP, H), jnp.float32)
)


def _fin_body(part_ref, degp_ref, b_ref, g_ref, be_ref,
              fc1w_ref, fc1b_ref, fc2w_ref, fc2b_ref, out_ref):
    dinv = _dinv_from(degp_ref[...])[:, None]
    h = _bn_relu(part_ref[...], dinv, b_ref[...], g_ref[...], be_ref[...])
    z = jnp.maximum(
        jnp.dot(h, fc1w_ref[...], preferred_element_type=jnp.float32,
                precision=_HI) + fc1b_ref[...][None, :],
        0.0,
    )
    o = (
        jnp.dot(z, fc2w_ref[...], preferred_element_type=jnp.float32,
                precision=_HI) + fc2b_ref[...][None, :]
    )
    m = jnp.max(o, axis=1, keepdims=True)
    lse = jnp.log(jnp.sum(jnp.exp(o - m), axis=1, keepdims=True)) + m
    out_ref[...] = o - lse


_fin = pl.pallas_call(
    _fin_body, out_shape=jax.ShapeDtypeStruct((N, 2), jnp.float32)
)


def kernel(x, edge_index, W0, b0, W1, b1, W2, b2, g0, be0, g1, be1, g2, be2,
           fc1_w, fc1_b, fc2_w, fc2_b):
    src = edge_index[0]
    dst = edge_index[1]
    src3 = src.reshape(NW, NCH, K)
    dst3 = dst.reshape(NW, NCH, K)
    zeros1 = jnp.zeros((RPS,), jnp.float32)
    zeros2 = jnp.zeros((RPS, H), jnp.float32)
    ones = jnp.ones((K,), jnp.float32)

    degp = _deg_kernel(dst3, zeros1, ones)
    hws = _tc0(degp, x, W0)
    for b, g, be, Wn in ((b0, g0, be0, W1), (b1, g1, be1, W2)):
        part = _scatter_kernel(hws, src3, dst3, zeros2)
        hws = _mid(part, degp, b, g, be, Wn)
    part = _scatter_kernel(hws, src3, dst3, zeros2)
    return _fin(part, degp, b2, g2, be2, fc1_w, fc1_b, fc2_w, fc2_b)
